# baseline (speedup 1.0000x reference)
import functools

import jax
import jax.numpy as jnp
from jax import lax
from jax.experimental import pallas as pl
from jax.experimental.pallas import tpu as pltpu
from jax.experimental.pallas import tpu_sc as plsc

D_MODEL = 768
MAX_LEN = 2048
SEQ = 512
BASE = MAX_LEN - SEQ
WIN = 1024

NUM_CORES = 2
NUM_SUBCORES = 16
STAGE_ROWS = WIN // NUM_SUBCORES
CHUNK = 16
CHUNKS_PER_ROW = SEQ // CHUNK
TCHUNK = 16
TCHUNKS_PER_ROW = SEQ // TCHUNK


@functools.partial(
    pl.kernel,
    mesh=plsc.VectorSubcoreMesh(core_axis_name="c", subcore_axis_name="s"),
    out_type=jax.ShapeDtypeStruct((SEQ, SEQ, D_MODEL), jnp.float32),
    scratch_types=[
        pltpu.VMEM_SHARED((2, WIN, D_MODEL), jnp.float32),
        pltpu.VMEM_SHARED((NUM_SUBCORES, CHUNK, D_MODEL), jnp.float32),
        pltpu.VMEM((TCHUNK, D_MODEL), jnp.float32),
        pltpu.SemaphoreType.DMA,
    ],
)
def _rpe_sc(shifted_hbm, out_hbm, stage, sbuf, buf, sem):
    c = lax.axis_index("c")
    s = lax.axis_index("s")

    for u in range(2):
        pltpu.sync_copy(
            shifted_hbm.at[2 * c + u, pl.ds(s * STAGE_ROWS, STAGE_ROWS)],
            stage.at[u, pl.ds(s * STAGE_ROWS, STAGE_ROWS)],
        )
    plsc.subcore_barrier()

    copies = []
    for q in range(4):
        m = s * 4 + q
        off = pl.multiple_of(504 - 8 * m, 8)
        for u in range(2):
            i = 8 * m + (7 - u) - 2 * c
            copies.append(
                pltpu.async_copy(stage.at[u, pl.ds(off, SEQ)], out_hbm.at[i], sem)
            )

    for q in range(4):
        m = s * 4 + q
        off = pl.multiple_of(504 - 8 * m, 8)
        for u in range(2):
            i = 8 * m + (3 - u) - 2 * c
            p = (4 + u) + 2 * c
            if u == 0 and q < 3:
                for kc in range(CHUNKS_PER_ROW):
                    j0 = kc * CHUNK
                    pltpu.sync_copy(shifted_hbm.at[p, pl.ds(off + j0, CHUNK)],
                                    sbuf.at[s])
                    pltpu.sync_copy(sbuf.at[s], out_hbm.at[i, pl.ds(j0, CHUNK)])
            else:
                for kc in range(TCHUNKS_PER_ROW):
                    j0 = kc * TCHUNK
                    pltpu.sync_copy(shifted_hbm.at[p, pl.ds(off + j0, TCHUNK)],
                                    buf)
                    pltpu.sync_copy(buf, out_hbm.at[i, pl.ds(j0, TCHUNK)])

    for cp in copies:
        cp.wait()


def kernel(rel_emb, length):
    del length
    shifted = jnp.stack(
        [lax.slice_in_dim(rel_emb, BASE + p, BASE + p + WIN, axis=0)
         for p in range(8)]
    )
    return _rpe_sc(shifted)


# speedup vs baseline: 1.4825x; 1.4825x over previous
import functools

import jax
import jax.numpy as jnp
from jax import lax
from jax.experimental import pallas as pl
from jax.experimental.pallas import tpu as pltpu
from jax.experimental.pallas import tpu_sc as plsc

D_MODEL = 768
MAX_LEN = 2048
SEQ = 512
BASE = MAX_LEN - SEQ
WIN = 1024

NUM_CORES = 2
NUM_SUBCORES = 16
STAGE_ROWS = WIN // NUM_SUBCORES
TCHUNK = 16
TCHUNKS_PER_ROW = SEQ // TCHUNK
NBUF = 2
BROWS = 8
NCHUNKS = BROWS * TCHUNKS_PER_ROW


@functools.partial(
    pl.kernel,
    mesh=plsc.VectorSubcoreMesh(core_axis_name="c", subcore_axis_name="s"),
    out_type=jax.ShapeDtypeStruct((SEQ, SEQ, D_MODEL), jnp.float32),
    scratch_types=[
        pltpu.VMEM_SHARED((2, WIN, D_MODEL), jnp.float32),
        pltpu.VMEM((NBUF, TCHUNK, D_MODEL), jnp.float32),
        pltpu.SemaphoreType.DMA,
        pltpu.SemaphoreType.DMA,
        pltpu.SemaphoreType.DMA,
        pltpu.SemaphoreType.DMA,
        pltpu.SemaphoreType.DMA,
    ],
)
def _rpe_sc(shifted_hbm, out_hbm, stage, buf, sem, si0, si1, so0, so1):
    c = lax.axis_index("c")
    s = lax.axis_index("s")
    sem_in = [si0, si1]
    sem_out = [so0, so1]

    for u in range(2):
        pltpu.sync_copy(
            shifted_hbm.at[2 * c + u, pl.ds(s * STAGE_ROWS, STAGE_ROWS)],
            stage.at[u, pl.ds(s * STAGE_ROWS, STAGE_ROWS)],
        )
    plsc.subcore_barrier()

    copies = []
    for q in range(4):
        m = s * 4 + q
        off = pl.multiple_of(504 - 8 * m, 8)
        for u in range(2):
            i = 8 * m + (7 - u) - 2 * c
            copies.append(
                pltpu.async_copy(stage.at[u, pl.ds(off, SEQ)], out_hbm.at[i], sem)
            )

    def chunk_coords(k):
        q, rest = divmod(k, 2 * TCHUNKS_PER_ROW)
        u, kc = divmod(rest, TCHUNKS_PER_ROW)
        m = s * 4 + q
        i = 8 * m + (3 - u) - 2 * c
        p = (4 + u) + 2 * c
        off = pl.multiple_of(504 - 8 * m, 8)
        j0 = kc * TCHUNK
        return i, p, off, j0

    in_cp = [None] * NBUF
    out_cp = [None] * NBUF
    for k in range(NCHUNKS + 1):
        if k < NCHUNKS:
            b = k % NBUF
            if out_cp[b] is not None:
                out_cp[b].wait()
            i, p, off, j0 = chunk_coords(k)
            in_cp[b] = pltpu.async_copy(
                shifted_hbm.at[p, pl.ds(off + j0, TCHUNK)], buf.at[b], sem_in[b])
        if k >= 1:
            kb = (k - 1) % NBUF
            in_cp[kb].wait()
            i, p, off, j0 = chunk_coords(k - 1)
            out_cp[kb] = pltpu.async_copy(
                buf.at[kb], out_hbm.at[i, pl.ds(j0, TCHUNK)], sem_out[kb])
    for b in range(NBUF):
        if out_cp[b] is not None:
            out_cp[b].wait()

    for cp in copies:
        cp.wait()


def kernel(rel_emb, length):
    del length
    shifted = jnp.stack(
        [lax.slice_in_dim(rel_emb, BASE + p, BASE + p + WIN, axis=0)
         for p in range(8)]
    )
    return _rpe_sc(shifted)


# speedup vs baseline: 1.4830x; 1.0003x over previous
import functools

import jax
import jax.numpy as jnp
from jax import lax
from jax.experimental import pallas as pl
from jax.experimental.pallas import tpu as pltpu
from jax.experimental.pallas import tpu_sc as plsc

D_MODEL = 768
MAX_LEN = 2048
SEQ = 512
BASE = MAX_LEN - SEQ
WIN = 1024

NUM_CORES = 2
NUM_SUBCORES = 16
STAGE_ROWS = WIN // NUM_SUBCORES
TCHUNK = 16
TCHUNKS_PER_ROW = SEQ // TCHUNK
NBUF = 2
BROWS = 8
NCHUNKS = BROWS * TCHUNKS_PER_ROW


@functools.partial(
    pl.kernel,
    mesh=plsc.VectorSubcoreMesh(core_axis_name="c", subcore_axis_name="s"),
    out_type=jax.ShapeDtypeStruct((SEQ, SEQ, D_MODEL), jnp.float32),
    scratch_types=[
        pltpu.VMEM_SHARED((2, WIN, D_MODEL), jnp.float32),
        pltpu.VMEM((NBUF, TCHUNK, D_MODEL), jnp.float32),
        pltpu.SemaphoreType.DMA,
        pltpu.SemaphoreType.DMA,
        pltpu.SemaphoreType.DMA,
        pltpu.SemaphoreType.DMA,
        pltpu.SemaphoreType.DMA,
        pltpu.SemaphoreType.DMA,
        pltpu.SemaphoreType.DMA,
        pltpu.SemaphoreType.DMA,
        pltpu.SemaphoreType.DMA,
    ],
)
def _rpe_sc(shifted_hbm, out_hbm, stage, buf, sem,
            si0, si1, si2, si3, so0, so1, so2, so3):
    c = lax.axis_index("c")
    s = lax.axis_index("s")
    sem_in = [si0, si1, si2, si3]
    sem_out = [so0, so1, so2, so3]

    for u in range(2):
        for h in range(2):
            r0 = s * STAGE_ROWS + h * (STAGE_ROWS // 2)
            pltpu.sync_copy(
                shifted_hbm.at[2 * c + u, pl.ds(r0, STAGE_ROWS // 2)],
                stage.at[u, pl.ds(r0, STAGE_ROWS // 2)],
            )
    plsc.subcore_barrier()

    copies = []
    for q in range(4):
        m = s * 4 + q
        off = pl.multiple_of(504 - 8 * m, 8)
        for u in range(2):
            i = 8 * m + (7 - u) - 2 * c
            copies.append(
                pltpu.async_copy(stage.at[u, pl.ds(off, SEQ)], out_hbm.at[i], sem)
            )

    def chunk_coords(k):
        q, rest = divmod(k, 2 * TCHUNKS_PER_ROW)
        u, kc = divmod(rest, TCHUNKS_PER_ROW)
        m = s * 4 + q
        i = 8 * m + (3 - u) - 2 * c
        p = (4 + u) + 2 * c
        off = pl.multiple_of(504 - 8 * m, 8)
        j0 = kc * TCHUNK
        return i, p, off, j0

    in_cp = [None] * NBUF
    out_cp = [None] * NBUF
    for k in range(NCHUNKS + 1):
        if k < NCHUNKS:
            b = k % NBUF
            if out_cp[b] is not None:
                out_cp[b].wait()
            i, p, off, j0 = chunk_coords(k)
            in_cp[b] = pltpu.async_copy(
                shifted_hbm.at[p, pl.ds(off + j0, TCHUNK)], buf.at[b], sem_in[b])
        if k >= 1:
            kb = (k - 1) % NBUF
            in_cp[kb].wait()
            i, p, off, j0 = chunk_coords(k - 1)
            out_cp[kb] = pltpu.async_copy(
                buf.at[kb], out_hbm.at[i, pl.ds(j0, TCHUNK)], sem_out[kb])
    for b in range(NBUF):
        if out_cp[b] is not None:
            out_cp[b].wait()

    for cp in copies:
        cp.wait()


def kernel(rel_emb, length):
    del length
    shifted = jnp.stack(
        [lax.slice_in_dim(rel_emb, BASE + p, BASE + p + WIN, axis=0)
         for p in range(8)]
    )
    return _rpe_sc(shifted)
